# initial kernel scaffold (unmeasured)
import jax
import jax.numpy as jnp
from jax import lax
from jax.experimental import pallas as pl
from jax.experimental.pallas import tpu as pltpu

N_DEV = 4


def _gelu(y):
    c = 0.7978845608028654
    return 0.5 * y * (1.0 + jnp.tanh(c * (y + 0.044715 * y * y * y)))


def kernel(x, w_mat):
    m_per, k = x.shape
    _, n_per = w_mat.shape

    def body(x_ref, w_ref, out_ref, comm_ref, send_sems, recv_sems):
        my = lax.axis_index("i")
        left = lax.rem(my - 1 + N_DEV, N_DEV)
        right = lax.rem(my + 1, N_DEV)

        barrier_sem = pltpu.get_barrier_semaphore()
        for nbr in (left, right):
            pl.semaphore_signal(
                barrier_sem, inc=1,
                device_id=(nbr,), device_id_type=pl.DeviceIdType.MESH,
            )
        pl.semaphore_wait(barrier_sem, 2)

        comm_ref[0] = x_ref[...]
        out_ref[pl.ds(my * m_per, m_per), :] = _gelu(
            jnp.dot(x_ref[...], w_ref[...], preferred_element_type=jnp.float32)
        )

        for h in range(N_DEV - 1):
            s, r = h % 2, (h + 1) % 2
            rdma = pltpu.make_async_remote_copy(
                src_ref=comm_ref.at[s],
                dst_ref=comm_ref.at[r],
                send_sem=send_sems.at[s],
                recv_sem=recv_sems.at[r],
                device_id=(right,),
                device_id_type=pl.DeviceIdType.MESH,
            )
            rdma.start()
            rdma.wait()

            origin = lax.rem(my - h - 1 + N_DEV, N_DEV)
            out_ref[pl.ds(origin * m_per, m_per), :] = _gelu(
                jnp.dot(comm_ref[r], w_ref[...],
                        preferred_element_type=jnp.float32)
            )

    return pl.pallas_call(
        body,
        out_shape=jax.ShapeDtypeStruct((N_DEV * m_per, n_per), jnp.float32),
        in_specs=[
            pl.BlockSpec(memory_space=pltpu.VMEM),
            pl.BlockSpec(memory_space=pltpu.VMEM),
        ],
        out_specs=pl.BlockSpec(memory_space=pltpu.VMEM),
        scratch_shapes=[
            pltpu.VMEM((2, m_per, k), x.dtype),
            pltpu.SemaphoreType.DMA((2,)),
            pltpu.SemaphoreType.DMA((2,)),
        ],
        compiler_params=pltpu.CompilerParams(collective_id=0),
    )(x, w_mat)


# baseline (device time: 353629 ns/iter reference)
import jax
import jax.numpy as jnp
from jax import lax
from jax.experimental import pallas as pl
from jax.experimental.pallas import tpu as pltpu

N_DEV = 4
ROW_TILE = 512


def _gelu(y):
    c = 0.7978845608028654
    return 0.5 * y * (1.0 + jnp.tanh(c * (y + 0.044715 * y * y * y)))


def kernel(x, w_mat):
    m_per, k = x.shape
    _, n_per = w_mat.shape
    n_tiles = m_per // ROW_TILE

    def body(x_ref, w_ref, out_ref, comm_ref, send_sems, recv_sems):
        my = lax.axis_index("i")
        left = lax.rem(my - 1 + N_DEV, N_DEV)
        right = lax.rem(my + 1, N_DEV)

        barrier_sem = pltpu.get_barrier_semaphore()
        for nbr in (left, right):
            pl.semaphore_signal(
                barrier_sem, inc=1,
                device_id=(nbr,), device_id_type=pl.DeviceIdType.MESH,
            )
        pl.semaphore_wait(barrier_sem, 2)

        comm_ref[0] = x_ref[...]

        def step(h, _):
            slot = lax.rem(h, 2)
            nslot = lax.rem(h + 1, 2)
            origin = lax.rem(my - h + N_DEV, N_DEV)
            rdma = pltpu.make_async_remote_copy(
                src_ref=comm_ref.at[slot],
                dst_ref=comm_ref.at[nslot],
                send_sem=send_sems.at[slot],
                recv_sem=recv_sems.at[nslot],
                device_id=(right,),
                device_id_type=pl.DeviceIdType.MESH,
            )

            @pl.when(h < N_DEV - 1)
            def _():
                rdma.start()

            def tile(r, _):
                acc = jnp.dot(
                    comm_ref[slot, pl.ds(r * ROW_TILE, ROW_TILE), :],
                    w_ref[...],
                    preferred_element_type=jnp.float32,
                )
                out_ref[pl.ds(origin * m_per + r * ROW_TILE, ROW_TILE), :] = (
                    _gelu(acc).astype(out_ref.dtype)
                )
                return 0

            lax.fori_loop(0, n_tiles, tile, 0)

            @pl.when(h < N_DEV - 1)
            def _():
                rdma.wait()

            return 0

        lax.fori_loop(0, N_DEV, step, 0)

    out = pl.pallas_call(
        body,
        out_shape=jax.ShapeDtypeStruct((N_DEV * m_per, n_per), jnp.bfloat16),
        in_specs=[
            pl.BlockSpec(memory_space=pltpu.VMEM),
            pl.BlockSpec(memory_space=pltpu.VMEM),
        ],
        out_specs=pl.BlockSpec(memory_space=pltpu.VMEM),
        scratch_shapes=[
            pltpu.VMEM((2, m_per, k), jnp.bfloat16),
            pltpu.SemaphoreType.DMA((2,)),
            pltpu.SemaphoreType.DMA((2,)),
        ],
        compiler_params=pltpu.CompilerParams(
            collective_id=0,
            vmem_limit_bytes=63 * 1024 * 1024,
        ),
    )(x.astype(jnp.bfloat16), w_mat.astype(jnp.bfloat16))
    return out


# device time: 218874 ns/iter; 1.6157x vs baseline; 1.6157x over previous
import jax
import jax.numpy as jnp
from jax import lax
from jax.experimental import pallas as pl
from jax.experimental.pallas import tpu as pltpu

N_DEV = 4


def _gelu(y):
    c = 0.7978845608028654
    return 0.5 * y * (1.0 + jnp.tanh(c * (y + 0.044715 * y * y * y)))


def kernel(x, w_mat):
    m_per, k = x.shape
    _, n_per = w_mat.shape
    half = m_per // 2

    def body(x_ref, w_ref, out_ref, comm_ref, send_sems, recv_sems):
        my = lax.axis_index("i")
        left = lax.rem(my - 1 + N_DEV, N_DEV)
        right = lax.rem(my + 1, N_DEV)

        barrier_sem = pltpu.get_barrier_semaphore()
        for nbr in (left, right):
            pl.semaphore_signal(
                barrier_sem, inc=1,
                device_id=(nbr,), device_id_type=pl.DeviceIdType.MESH,
            )
        pl.semaphore_wait(barrier_sem, 2)

        comm_ref[0, 0] = x_ref[pl.ds(0, half), :]
        comm_ref[0, 1] = x_ref[pl.ds(half, half), :]

        def step(h, _):
            slot = lax.rem(h, 2)
            nslot = lax.rem(h + 1, 2)

            def make_rdma(d, dst_dev):
                return pltpu.make_async_remote_copy(
                    src_ref=comm_ref.at[slot, d],
                    dst_ref=comm_ref.at[nslot, d],
                    send_sem=send_sems.at[slot, d],
                    recv_sem=recv_sems.at[nslot, d],
                    device_id=(dst_dev,),
                    device_id_type=pl.DeviceIdType.MESH,
                )

            rdma_cw = make_rdma(0, right)
            rdma_ccw = make_rdma(1, left)

            @pl.when(h < N_DEV - 1)
            def _():
                rdma_cw.start()
                rdma_ccw.start()

            def dir_compute(d, _):
                origin = lax.rem(my + (2 * d - 1) * h + 2 * N_DEV, N_DEV)
                acc = jnp.dot(
                    comm_ref[slot, d],
                    w_ref[...],
                    preferred_element_type=jnp.float32,
                )
                out_ref[pl.ds(origin * m_per + d * half, half), :] = (
                    _gelu(acc).astype(out_ref.dtype)
                )
                return 0

            lax.fori_loop(0, 2, dir_compute, 0)

            @pl.when(h < N_DEV - 1)
            def _():
                rdma_cw.wait()
                rdma_ccw.wait()

            return 0

        lax.fori_loop(0, N_DEV, step, 0)

    out = pl.pallas_call(
        body,
        out_shape=jax.ShapeDtypeStruct((N_DEV * m_per, n_per), jnp.bfloat16),
        in_specs=[
            pl.BlockSpec(memory_space=pltpu.VMEM),
            pl.BlockSpec(memory_space=pltpu.VMEM),
        ],
        out_specs=pl.BlockSpec(memory_space=pltpu.VMEM),
        scratch_shapes=[
            pltpu.VMEM((2, 2, half, k), jnp.bfloat16),
            pltpu.SemaphoreType.DMA((2, 2)),
            pltpu.SemaphoreType.DMA((2, 2)),
        ],
        compiler_params=pltpu.CompilerParams(
            collective_id=0,
            vmem_limit_bytes=63 * 1024 * 1024,
        ),
    )(x.astype(jnp.bfloat16), w_mat.astype(jnp.bfloat16))
    return out


# device time: 183649 ns/iter; 1.9256x vs baseline; 1.1918x over previous
import jax
import jax.numpy as jnp
from jax import lax
from jax.experimental import pallas as pl
from jax.experimental.pallas import tpu as pltpu

N_DEV = 4
W_TILE = 512


def _gelu(y):
    c = 0.7978845608028654
    return 0.5 * y * (1.0 + jnp.tanh(c * (y + 0.044715 * y * y * y)))


def kernel(x, w_mat):
    m_per, k = x.shape
    _, n_per = w_mat.shape
    half = m_per // 2
    n_wtiles = k // W_TILE

    def body(x_ref, w_hbm, out_hbm, comm_ref, wbf_ref, wstage_ref,
             ostage_ref, send_sems, recv_sems, wload_sem, ocopy_sems):
        my = lax.axis_index("i")
        left = lax.rem(my - 1 + N_DEV, N_DEV)
        right = lax.rem(my + 1, N_DEV)

        barrier_sem = pltpu.get_barrier_semaphore()
        for nbr in (left, right):
            pl.semaphore_signal(
                barrier_sem, inc=1,
                device_id=(nbr,), device_id_type=pl.DeviceIdType.MESH,
            )
        pl.semaphore_wait(barrier_sem, 2)

        comm_ref[0, 0] = x_ref[pl.ds(0, half), :].astype(jnp.bfloat16)
        comm_ref[0, 1] = x_ref[pl.ds(half, half), :].astype(jnp.bfloat16)

        def w_tile_copy(kt):
            return pltpu.make_async_copy(
                w_hbm.at[pl.ds(kt * W_TILE, W_TILE)],
                wstage_ref,
                wload_sem,
            )

        def out_copy(origin, d):
            return pltpu.make_async_copy(
                ostage_ref.at[pl.ds(d * half, half)],
                out_hbm.at[pl.ds(origin * m_per + d * half, half)],
                ocopy_sems.at[d],
            )

        def origin_of(h, d):
            return lax.rem(my + (2 * d - 1) * h + 2 * N_DEV, N_DEV)

        def step(h, _):
            slot = lax.rem(h, 2)
            nslot = lax.rem(h + 1, 2)

            def make_rdma(d, dst_dev):
                return pltpu.make_async_remote_copy(
                    src_ref=comm_ref.at[slot, d],
                    dst_ref=comm_ref.at[nslot, d],
                    send_sem=send_sems.at[slot, d],
                    recv_sem=recv_sems.at[nslot, d],
                    device_id=(dst_dev,),
                    device_id_type=pl.DeviceIdType.MESH,
                )

            rdma_cw = make_rdma(0, right)
            rdma_ccw = make_rdma(1, left)

            @pl.when(h < N_DEV - 1)
            def _():
                rdma_cw.start()
                rdma_ccw.start()

            @pl.when(h == 0)
            def _():
                def wconv(kt, _):
                    w_tile_copy(kt).start()
                    w_tile_copy(kt).wait()
                    wbf_ref[pl.ds(kt * W_TILE, W_TILE), :] = (
                        wstage_ref[...].astype(jnp.bfloat16)
                    )
                    return 0

                lax.fori_loop(0, n_wtiles, wconv, 0)

            def dir_compute(d, _):
                acc = jnp.dot(
                    comm_ref[slot, d],
                    wbf_ref[...],
                    preferred_element_type=jnp.float32,
                )
                ostage_ref[pl.ds(d * half, half), :] = (
                    _gelu(acc).astype(jnp.bfloat16)
                )
                out_copy(origin_of(h, d), d).start()
                return 0

            lax.fori_loop(0, 2, dir_compute, 0)

            @pl.when(h < N_DEV - 1)
            def _():
                rdma_cw.wait()
                rdma_ccw.wait()

            def ocopy_wait(d, _):
                out_copy(origin_of(h, d), d).wait()
                return 0

            lax.fori_loop(0, 2, ocopy_wait, 0)
            return 0

        lax.fori_loop(0, N_DEV, step, 0)

    out = pl.pallas_call(
        body,
        out_shape=jax.ShapeDtypeStruct((N_DEV * m_per, n_per), jnp.bfloat16),
        in_specs=[
            pl.BlockSpec(memory_space=pltpu.VMEM),
            pl.BlockSpec(memory_space=pl.ANY),
        ],
        out_specs=pl.BlockSpec(memory_space=pl.ANY),
        scratch_shapes=[
            pltpu.VMEM((2, 2, half, k), jnp.bfloat16),
            pltpu.VMEM((k, n_per), jnp.bfloat16),
            pltpu.VMEM((W_TILE, n_per), jnp.float32),
            pltpu.VMEM((m_per, n_per), jnp.bfloat16),
            pltpu.SemaphoreType.DMA((2, 2)),
            pltpu.SemaphoreType.DMA((2, 2)),
            pltpu.SemaphoreType.DMA,
            pltpu.SemaphoreType.DMA((2,)),
        ],
        compiler_params=pltpu.CompilerParams(
            collective_id=0,
            vmem_limit_bytes=63 * 1024 * 1024,
        ),
    )(x, w_mat)
    return out
